# Initial kernel scaffold; baseline (speedup 1.0000x reference)
import jax
import jax.numpy as jnp
from jax.experimental import pallas as pl


def kernel(x, pos, params, edge_index, batch):
    raise NotImplementedError("write your pallas kernel here")


# speedup vs baseline: 2.6629x; 2.6629x over previous
import math
import functools

import jax
import jax.numpy as jnp
from jax.experimental import pallas as pl

_B = 8
_NPER0 = 1250
_HID = 64
_RATIO = 0.5
_BIG = 1e30


def _knn_kernel(p_ref, o_ref, *, P, m, k):
    p = p_ref[0]
    dx = p[0][:, None] - p[0][None, :]
    dy = p[1][:, None] - p[1][None, :]
    dz = p[2][:, None] - p[2][None, :]
    d = dx * dx + dy * dy + dz * dz
    col = jax.lax.broadcasted_iota(jnp.int32, (P, P), 1)
    row = jax.lax.broadcasted_iota(jnp.int32, (P, P), 0)
    d = jnp.where((col == row) | (col >= m), _BIG, d)
    rows = []
    for _ in range(k):
        mn = jnp.min(d, axis=1, keepdims=True)
        idx = jnp.min(jnp.where(d == mn, col, jnp.int32(2**30)), axis=1)
        rows.append(idx)
        d = jnp.where(col == idx[:, None], _BIG, d)
    for _ in range(k, 8):
        rows.append(rows[-1])
    o_ref[0] = jnp.stack(rows, axis=0)


def _knn_pallas(pos, nb, m, k):
    P = ((m + 127) // 128) * 128
    posb = pos.reshape(nb, m, 3)
    posb = jnp.pad(posb, ((0, 0), (0, P - m), (0, 0)))
    pT = jnp.swapaxes(posb, 1, 2)
    pT = jnp.pad(pT, ((0, 0), (0, 5), (0, 0)))
    out = pl.pallas_call(
        functools.partial(_knn_kernel, P=P, m=m, k=k),
        grid=(nb,),
        in_specs=[pl.BlockSpec((1, 8, P), lambda b: (b, 0, 0))],
        out_specs=pl.BlockSpec((1, 8, P), lambda b: (b, 0, 0)),
        out_shape=jax.ShapeDtypeStruct((nb, 8, P), jnp.int32),
    )(pT)
    nbr = out[:, :k, :m]
    offs = (jnp.arange(nb, dtype=jnp.int32) * m)[:, None, None]
    return jnp.swapaxes(nbr + offs, 1, 2).reshape(nb * m, k)


def _graph_conv_seg(x, src, dst, n, W_rel, b_rel, W_root):
    msg = x[src]
    s = jax.ops.segment_sum(msg, dst, num_segments=n)
    cnt = jax.ops.segment_sum(jnp.ones((src.shape[0],), x.dtype), dst,
                              num_segments=n)
    mean = s / jnp.clip(cnt, 1.0)[:, None]
    return mean @ W_rel + b_rel + x @ W_root


def _seg_softmax(scores, seg, n):
    m = jax.ops.segment_max(scores, seg, num_segments=n)
    m = jnp.where(jnp.isfinite(m), m, 0.0)
    e = jnp.exp(scores - m[seg])
    d = jax.ops.segment_sum(e, seg, num_segments=n)
    return e / (d[seg] + 1e-16)


def _asap_pool_seg(x, src, dst, n, n_per, p, i):
    loop = jnp.arange(n)
    src2 = jnp.concatenate([src, loop])
    dst2 = jnp.concatenate([dst, loop])
    x_pool_j = x[src2]
    x_q = jax.ops.segment_max(x_pool_j, dst2, num_segments=n)
    x_q = jnp.where(jnp.isfinite(x_q), x_q, 0.0)
    x_q = x_q @ p['pool%d_lin_W' % i] + p['pool%d_lin_b' % i]
    score = (jnp.concatenate([x_q[dst2], x_pool_j], axis=-1)
             @ p['pool%d_att_W' % i] + p['pool%d_att_b' % i])
    score = jax.nn.leaky_relu(score[:, 0], 0.2)
    score = _seg_softmax(score, dst2, n)
    v_j = x[src2] * score[:, None]
    x_new = jax.ops.segment_sum(v_j, dst2, num_segments=n)
    a = x_new @ p['pool%d_gs1_W' % i]
    b = x_new @ p['pool%d_gs2_W' % i]
    agg = jax.ops.segment_sum(a[src2] - b[dst2], dst2, num_segments=n)
    fitness = jax.nn.sigmoid(
        (agg + x_new @ p['pool%d_gs3_W' % i] + p['pool%d_gs3_b' % i])[:, 0])
    return _topk_select(x_new, fitness, n, n_per)


def _topk_select(x_new, fitness, n, n_per):
    nb = n // n_per
    m = int(math.ceil(_RATIO * n_per))
    f2 = fitness.reshape(nb, n_per)
    idx = jnp.argsort(-f2, axis=1)[:, :m]
    perm = (idx + jnp.arange(nb)[:, None] * n_per).reshape(-1)
    x_out = x_new[perm] * fitness[perm][:, None]
    return x_out, perm, m


def _conv_knn(x, nbr, W_rel, b_rel, W_root):
    g = x[nbr]
    mean = g.mean(axis=1)
    return mean @ W_rel + b_rel + x @ W_root


def _pool_knn(x, nbr, n_per, p, i):
    n, k = nbr.shape
    H = x.shape[1]
    nbrs2 = jnp.concatenate(
        [nbr, jnp.arange(n, dtype=nbr.dtype)[:, None]], axis=1)
    G = x[nbrs2]
    x_q = G.max(axis=1) @ p['pool%d_lin_W' % i] + p['pool%d_lin_b' % i]
    att_W = p['pool%d_att_W' % i]
    s1 = (x_q @ att_W[:H])[:, 0]
    s2 = (x @ att_W[H:])[:, 0]
    score = s1[:, None] + s2[nbrs2] + p['pool%d_att_b' % i][0]
    score = jax.nn.leaky_relu(score, 0.2)
    mx = jnp.max(score, axis=1, keepdims=True)
    e = jnp.exp(score - mx)
    w = e / (jnp.sum(e, axis=1, keepdims=True) + 1e-16)
    x_new = jnp.sum(G * w[:, :, None], axis=1)
    a = (x_new @ p['pool%d_gs1_W' % i])[:, 0]
    b = (x_new @ p['pool%d_gs2_W' % i])[:, 0]
    agg = jnp.sum(a[nbrs2], axis=1) - (k + 1) * b
    fitness = jax.nn.sigmoid(
        agg + (x_new @ p['pool%d_gs3_W' % i])[:, 0]
        + p['pool%d_gs3_b' % i][0])
    return _topk_select(x_new, fitness, n, n_per)


def kernel(x, pos, params, edge_index, batch):
    src, dst = edge_index[0], edge_index[1]
    n = x.shape[0]
    n_per = _NPER0
    xs = []

    x = jax.nn.relu(_graph_conv_seg(x, src, dst, n,
                                    params['conv0_rel_W'],
                                    params['conv0_rel_b'],
                                    params['conv0_root_W']))
    x, perm, m = _asap_pool_seg(x, src, dst, n, n_per, params, 0)
    n_per = m
    n = x.shape[0]
    xs.append(jnp.max(x.reshape(_B, n_per, _HID), axis=1))
    pos = pos[perm]
    nbr = _knn_pallas(pos, _B, n_per, 6)

    for i in (1, 2):
        x = jax.nn.relu(_conv_knn(x, nbr,
                                  params['conv%d_rel_W' % i],
                                  params['conv%d_rel_b' % i],
                                  params['conv%d_root_W' % i]))
        x, perm, m = _pool_knn(x, nbr, n_per, params, i)
        n_per = m
        n = x.shape[0]
        xs.append(jnp.max(x.reshape(_B, n_per, _HID), axis=1))
        pos = pos[perm]
        if i < 2:
            nbr = _knn_pallas(pos, _B, n_per, 6 + 2 * i)

    h = jnp.concatenate(xs, axis=-1)
    h = jax.nn.relu(h @ params['lin1_W'] + params['lin1_b'])
    return h @ params['lin2_W'] + params['lin2_b']
